# baseline (speedup 1.0000x reference)
import functools

import jax
import jax.numpy as jnp
from jax import lax
from jax.experimental import pallas as pl
from jax.experimental.pallas import tpu as pltpu
from jax.experimental.pallas import tpu_sc as plsc

B = 16384
D = 64
V = 1000000
L = 16
NC, NS = 2, 16
NW = NC * NS
RPW = B // NW
CH = 128
CB = 32768

M_SC = 12
SCC = CB * M_SC
VTC = V - SCC
CPW = SCC // NW
SL = 128
NSL = CPW // SL

_mesh = plsc.VectorSubcoreMesh(core_axis_name="c", subcore_axis_name="s")


def _wsum_body(ut_ref, it_ref, wu_ref, wi_ref, vu_ref, vi_ref):
    vu_ref[0, :] = jnp.sum(ut_ref[...] * wu_ref[...], axis=0)
    vi_ref[0, :] = jnp.sum(it_ref[...] * wi_ref[...], axis=0)


_NCB = (VTC + CB - 1) // CB

_wsum = pl.pallas_call(
    _wsum_body,
    grid=(_NCB,),
    in_specs=[
        pl.BlockSpec((D, CB), lambda j: (0, M_SC + j)),
        pl.BlockSpec((D, CB), lambda j: (0, M_SC + j)),
        pl.BlockSpec((D, 1), lambda j: (0, 0)),
        pl.BlockSpec((D, 1), lambda j: (0, 0)),
    ],
    out_specs=[
        pl.BlockSpec((1, CB), lambda j: (0, j)),
        pl.BlockSpec((1, CB), lambda j: (0, j)),
    ],
    out_shape=[
        jax.ShapeDtypeStruct((1, VTC), jnp.float32),
        jax.ShapeDtypeStruct((1, VTC), jnp.float32),
    ],
)


@functools.partial(
    pl.kernel,
    mesh=_mesh,
    out_type=[
        jax.ShapeDtypeStruct((SCC,), jnp.float32),
        jax.ShapeDtypeStruct((SCC,), jnp.float32),
    ],
    scratch_types=[
        pltpu.VMEM((D, SL), jnp.float32),
        pltpu.VMEM((D, SL), jnp.float32),
        pltpu.VMEM((D, SL), jnp.float32),
        pltpu.VMEM((D, SL), jnp.float32),
        pltpu.VMEM((2 * D, L), jnp.float32),
        pltpu.VMEM((CPW,), jnp.float32),
        pltpu.VMEM((CPW,), jnp.float32),
        pltpu.SemaphoreType.DMA,
        pltpu.SemaphoreType.DMA,
    ],
)
def _vsweep_sc(utab, itab, wbb, vu_out, vi_out,
               ub0, ub1, ib0, ib1, wbb_v, vu_v, vi_v, sem0, sem1):
    wid = lax.axis_index("s") * NC + lax.axis_index("c")
    cbase = wid * CPW
    pltpu.sync_copy(wbb, wbb_v)

    bufs = [(ub0, ib0, sem0), (ub1, ib1, sem1)]

    def fire(s, buf):
        ub, ib, sm = bufs[buf]
        off = pl.multiple_of(cbase + s * SL, SL)
        pltpu.async_copy(utab.at[:, pl.ds(off, SL)], ub, sm)
        pltpu.async_copy(itab.at[:, pl.ds(off, SL)], ib, sm)

    def drain(buf):
        ub, ib, sm = bufs[buf]
        pltpu.make_async_copy(utab.at[:, pl.ds(0, SL)], ub, sm).wait()
        pltpu.make_async_copy(itab.at[:, pl.ds(0, SL)], ib, sm).wait()

    def compute(s, buf):
        ub, ib, _ = bufs[buf]
        accu = [None] * (SL // L)
        acci = [None] * (SL // L)
        for d in range(D):
            wu_d = wbb_v[d, :]
            wi_d = wbb_v[D + d, :]
            for g in range(SL // L):
                pu = ub[d, pl.ds(g * L, L)] * wu_d
                pi = ib[d, pl.ds(g * L, L)] * wi_d
                accu[g] = pu if accu[g] is None else accu[g] + pu
                acci[g] = pi if acci[g] is None else acci[g] + pi
        for g in range(SL // L):
            vu_v[pl.ds(s * SL + g * L, L)] = accu[g]
            vi_v[pl.ds(s * SL + g * L, L)] = acci[g]

    fire(0, 0)

    def step(h, carry):
        s0 = 2 * h
        fire(s0 + 1, 1)
        drain(0)
        compute(s0, 0)

        @pl.when(s0 + 2 < NSL)
        def _():
            fire(s0 + 2, 0)

        drain(1)
        compute(s0 + 1, 1)
        return carry

    lax.fori_loop(0, NSL // 2, step, 0)
    pltpu.sync_copy(vu_v, vu_out.at[pl.ds(cbase, CPW)])
    pltpu.sync_copy(vi_v, vi_out.at[pl.ds(cbase, CPW)])


@functools.partial(
    pl.kernel,
    mesh=_mesh,
    out_type=jax.ShapeDtypeStruct((B,), jnp.float32),
    scratch_types=[
        pltpu.VMEM((RPW,), jnp.int32),
        pltpu.VMEM((RPW,), jnp.int32),
        pltpu.VMEM((RPW,), jnp.int32),
        pltpu.VMEM((RPW,), jnp.int32),
        pltpu.VMEM((RPW,), jnp.int32),
        pltpu.VMEM((RPW,), jnp.int32),
        pltpu.VMEM((RPW,), jnp.float32),
        pltpu.VMEM((RPW,), jnp.float32),
        pltpu.VMEM((RPW,), jnp.float32),
        pltpu.VMEM((RPW,), jnp.float32),
        pltpu.VMEM((L,), jnp.float32),
        pltpu.VMEM((RPW,), jnp.float32),
        pltpu.SemaphoreType.DMA,
    ],
)
def _gather_sc(uids, iids, vu_sc, vi_sc, vu_tc, vi_tc, bvec, out,
               uidx_v, iidx_v, usc_i, utc_i, isc_i, itc_i,
               gu_sc, gu_tc, gi_sc, gi_tc, b_v, out_v, sem):
    wid = lax.axis_index("s") * NC + lax.axis_index("c")
    base = wid * RPW
    pltpu.sync_copy(uids.at[pl.ds(base, RPW)], uidx_v)
    pltpu.sync_copy(iids.at[pl.ds(base, RPW)], iidx_v)
    pltpu.sync_copy(bvec, b_v)
    for k in range(RPW // L):
        sl = pl.ds(k * L, L)
        uu = uidx_v[sl]
        ii = iidx_v[sl]
        usc_i[sl] = jnp.minimum(uu, SCC - 1)
        utc_i[sl] = jnp.maximum(uu - SCC, 0)
        isc_i[sl] = jnp.minimum(ii, SCC - 1)
        itc_i[sl] = jnp.maximum(ii - SCC, 0)
    handles = []
    for c in range(RPW // CH):
        sl = pl.ds(c * CH, CH)
        handles.append(pltpu.async_copy(
            vu_sc.at[usc_i.at[sl]], gu_sc.at[sl], sem))
        handles.append(pltpu.async_copy(
            vu_tc.at[utc_i.at[sl]], gu_tc.at[sl], sem))
        handles.append(pltpu.async_copy(
            vi_sc.at[isc_i.at[sl]], gi_sc.at[sl], sem))
        handles.append(pltpu.async_copy(
            vi_tc.at[itc_i.at[sl]], gi_tc.at[sl], sem))
    for h in handles:
        h.wait()
    bias = b_v[...]
    for k in range(RPW // L):
        sl = pl.ds(k * L, L)
        gu = jnp.where(uidx_v[sl] < SCC, gu_sc[sl], gu_tc[sl])
        gi = jnp.where(iidx_v[sl] < SCC, gi_sc[sl], gi_tc[sl])
        out_v[sl] = gu + gi + bias
    pltpu.sync_copy(out_v, out.at[pl.ds(base, RPW)])


def kernel(user_ids, item_ids, user_table, item_table, fc_w, fc_b):
    ut_t = user_table.T
    it_t = item_table.T
    wu = fc_w[:D]
    wi = fc_w[D:]
    wbb = jnp.broadcast_to(fc_w[:, 0][:, None], (2 * D, L))
    vu_sc, vi_sc = _vsweep_sc(ut_t, it_t, wbb)
    vu_tc, vi_tc = _wsum(ut_t, it_t, wu, wi)
    bvec = jnp.broadcast_to(fc_b, (L,))
    out = _gather_sc(user_ids, item_ids, vu_sc, vi_sc,
                     vu_tc.reshape(VTC), vi_tc.reshape(VTC), bvec)
    return out.reshape(B, 1)


# speedup vs baseline: 1.6088x; 1.6088x over previous
import functools

import jax
import jax.numpy as jnp
from jax import lax
from jax.experimental import pallas as pl
from jax.experimental.pallas import tpu as pltpu
from jax.experimental.pallas import tpu_sc as plsc

B = 16384
D = 64
V = 1000000
L = 16
NC, NS = 2, 16
NW = NC * NS
RPW = B // NW
CH = 128
CB = 16384

_mesh = plsc.VectorSubcoreMesh(core_axis_name="c", subcore_axis_name="s")


def _wsum_body(ut_ref, it_ref, wu_ref, wi_ref, vu_ref, vi_ref):
    vu_ref[0, :] = jnp.sum(ut_ref[...] * wu_ref[...], axis=0)
    vi_ref[0, :] = jnp.sum(it_ref[...] * wi_ref[...], axis=0)


_NCB = (V + CB - 1) // CB

_wsum = pl.pallas_call(
    _wsum_body,
    grid=(_NCB,),
    in_specs=[
        pl.BlockSpec((D, CB), lambda j: (0, j)),
        pl.BlockSpec((D, CB), lambda j: (0, j)),
        pl.BlockSpec((D, 1), lambda j: (0, 0)),
        pl.BlockSpec((D, 1), lambda j: (0, 0)),
    ],
    out_specs=[
        pl.BlockSpec((1, CB), lambda j: (0, j)),
        pl.BlockSpec((1, CB), lambda j: (0, j)),
    ],
    out_shape=[
        jax.ShapeDtypeStruct((1, V), jnp.float32),
        jax.ShapeDtypeStruct((1, V), jnp.float32),
    ],
)


@functools.partial(
    pl.kernel,
    mesh=_mesh,
    out_type=jax.ShapeDtypeStruct((B,), jnp.float32),
    scratch_types=[
        pltpu.VMEM((RPW,), jnp.int32),
        pltpu.VMEM((RPW,), jnp.int32),
        pltpu.VMEM((RPW,), jnp.float32),
        pltpu.VMEM((RPW,), jnp.float32),
        pltpu.VMEM((L,), jnp.float32),
        pltpu.VMEM((RPW,), jnp.float32),
        pltpu.SemaphoreType.DMA,
    ],
)
def _gather_sc(uids, iids, vu, vi, bvec, out,
               uidx_v, iidx_v, gu_v, gi_v, b_v, out_v, sem):
    wid = lax.axis_index("s") * NC + lax.axis_index("c")
    base = wid * RPW
    pltpu.sync_copy(uids.at[pl.ds(base, RPW)], uidx_v)
    pltpu.sync_copy(iids.at[pl.ds(base, RPW)], iidx_v)
    pltpu.sync_copy(bvec, b_v)
    handles = []
    for c in range(RPW // CH):
        handles.append(pltpu.async_copy(
            vu.at[uidx_v.at[pl.ds(c * CH, CH)]],
            gu_v.at[pl.ds(c * CH, CH)], sem))
        handles.append(pltpu.async_copy(
            vi.at[iidx_v.at[pl.ds(c * CH, CH)]],
            gi_v.at[pl.ds(c * CH, CH)], sem))
    for h in handles:
        h.wait()
    bias = b_v[...]
    for k in range(RPW // L):
        out_v[pl.ds(k * L, L)] = (gu_v[pl.ds(k * L, L)]
                                  + gi_v[pl.ds(k * L, L)] + bias)
    pltpu.sync_copy(out_v, out.at[pl.ds(base, RPW)])


def kernel(user_ids, item_ids, user_table, item_table, fc_w, fc_b):
    ut_t = user_table.T
    it_t = item_table.T
    wu = fc_w[:D]
    wi = fc_w[D:]
    vu, vi = _wsum(ut_t, it_t, wu, wi)
    bvec = jnp.broadcast_to(fc_b, (L,))
    out = _gather_sc(user_ids, item_ids, vu.reshape(V), vi.reshape(V), bvec)
    return out.reshape(B, 1)
